# Initial kernel scaffold; baseline (speedup 1.0000x reference)
import jax
import jax.numpy as jnp
from jax.experimental import pallas as pl


def kernel(features, mask, edge_index, W1, b1, W2, b2):
    raise NotImplementedError("write your pallas kernel here")


# speedup vs baseline: 3.1417x; 3.1417x over previous
import functools

import jax
import jax.numpy as jnp
from jax import lax
from jax.experimental import pallas as pl
from jax.experimental.pallas import tpu as pltpu
from jax.experimental.pallas import tpu_sc as plsc

N_NODES = 10000
FEATS = 128
NC = 2
NS = 16
NW = NC * NS
LANES = 16
CHUNK = 128
N_PAD = 10240
ROWS_PER_TILE = N_PAD // NS
ROW_COPIES = ROWS_PER_TILE // CHUNK

_MESH = plsc.VectorSubcoreMesh(core_axis_name="c", subcore_axis_name="s")


def _make_deg_kernel(e_pad):
    cpt = e_pad // (NS * CHUNK)

    @functools.partial(
        pl.kernel,
        out_type=jax.ShapeDtypeStruct((NC, N_PAD, FEATS), jnp.float32),
        mesh=_MESH,
        scratch_types=[
            pltpu.VMEM((CHUNK,), jnp.int32),
            pltpu.VMEM((CHUNK, FEATS), jnp.float32),
            pltpu.VMEM((CHUNK, FEATS), jnp.float32),
            pltpu.VMEM_SHARED((N_PAD, FEATS), jnp.float32),
        ],
    )
    def deg_kernel(src_hbm, dst_hbm, ones_hbm, zeros_hbm, out_hbm,
                   idx_v, ones_v, stage_v, acc_sp):
        c = lax.axis_index("c")
        s = lax.axis_index("s")
        row0 = s * ROWS_PER_TILE

        pltpu.sync_copy(ones_hbm, ones_v)
        pltpu.sync_copy(zeros_hbm, stage_v)
        for k in range(ROW_COPIES):
            pltpu.sync_copy(stage_v, acc_sp.at[pl.ds(row0 + k * CHUNK, CHUNK)])
        plsc.subcore_barrier()

        base = s * cpt * CHUNK

        def count(idx_hbm):
            def body(k, _):
                e0 = pl.multiple_of(base + k * CHUNK, CHUNK)
                pltpu.sync_copy(idx_hbm.at[pl.ds(e0, CHUNK)], idx_v)
                pltpu.sync_copy(ones_v, acc_sp.at[idx_v], add=True)
                return 0
            lax.fori_loop(0, cpt, body, 0)

        @pl.when(c == 0)
        def _():
            count(src_hbm)

        @pl.when(c == 1)
        def _():
            count(dst_hbm)

        plsc.subcore_barrier()
        for k in range(ROW_COPIES):
            r = row0 + k * CHUNK
            pltpu.sync_copy(acc_sp.at[pl.ds(r, CHUNK)], stage_v)
            pltpu.sync_copy(stage_v, out_hbm.at[c, pl.ds(r, CHUNK)])

    return deg_kernel


def _make_agg_kernel(e_pad):
    cpt = e_pad // (NW * CHUNK)

    @functools.partial(
        pl.kernel,
        out_type=jax.ShapeDtypeStruct((NC, N_PAD, FEATS), jnp.float32),
        mesh=_MESH,
        scratch_types=[
            pltpu.VMEM((CHUNK,), jnp.int32),
            pltpu.VMEM((CHUNK,), jnp.int32),
            pltpu.VMEM((CHUNK, FEATS), jnp.float32),
            pltpu.VMEM_SHARED((N_PAD, FEATS), jnp.float32),
            pltpu.SemaphoreType.DMA,
        ],
    )
    def agg_kernel(h_hbm, src_hbm, dst_hbm, zeros_hbm, out_hbm,
                   sidx_v, didx_v, rows_v, acc_sp, sem):
        c = lax.axis_index("c")
        s = lax.axis_index("s")
        row0 = s * ROWS_PER_TILE

        pltpu.sync_copy(zeros_hbm, rows_v)
        for k in range(ROW_COPIES):
            pltpu.sync_copy(rows_v, acc_sp.at[pl.ds(row0 + k * CHUNK, CHUNK)])
        plsc.subcore_barrier()

        wid = s * NC + c
        base = wid * cpt * CHUNK

        def body(k, _):
            e0 = pl.multiple_of(base + k * CHUNK, CHUNK)
            pltpu.sync_copy(src_hbm.at[pl.ds(e0, CHUNK)], sidx_v)
            pltpu.sync_copy(dst_hbm.at[pl.ds(e0, CHUNK)], didx_v)
            pltpu.async_copy(h_hbm.at[sidx_v], rows_v, sem).wait()
            pltpu.sync_copy(rows_v, acc_sp.at[didx_v], add=True)
            return 0

        lax.fori_loop(0, cpt, body, 0)
        plsc.subcore_barrier()

        for k in range(ROW_COPIES):
            r = row0 + k * CHUNK
            pltpu.sync_copy(acc_sp.at[pl.ds(r, CHUNK)], rows_v)
            pltpu.sync_copy(rows_v, out_hbm.at[c, pl.ds(r, CHUNK)])

    return agg_kernel


_BLK = 256
_GRID = N_PAD // _BLK


def _norm_src(deg_ref):
    return lax.rsqrt(jnp.maximum(deg_ref[0, :, :1], 1.0))


def _norm_dst(deg_ref):
    return lax.rsqrt(jnp.maximum(deg_ref[1, :, :1], 1.0))


def _prep_body(f_ref, m_ref, deg_ref, o_ref):
    o_ref[...] = f_ref[...] * m_ref[...] * _norm_src(deg_ref)


def _layer1_body(agg_ref, deg_ref, w_ref, b_ref, o_ref):
    agg = (agg_ref[0] + agg_ref[1]) * _norm_dst(deg_ref)
    z = jnp.dot(agg, w_ref[...], preferred_element_type=jnp.float32)
    h = jnp.maximum(z + b_ref[...], 0.0) * _norm_src(deg_ref)
    i = pl.program_id(0)
    row = i * _BLK + lax.broadcasted_iota(jnp.int32, (_BLK, 1), 0)
    o_ref[...] = jnp.where(row < N_NODES, h, 0.0)


def _layer2_body(agg_ref, deg_ref, w_ref, b_ref, o_ref):
    agg = (agg_ref[0] + agg_ref[1]) * _norm_dst(deg_ref)
    z = jnp.dot(agg, w_ref[...], preferred_element_type=jnp.float32)
    o_ref[...] = z + b_ref[...]


_row_spec = pl.BlockSpec((_BLK, FEATS), lambda i: (i, 0))
_part_spec = pl.BlockSpec((NC, _BLK, FEATS), lambda i: (0, i, 0))
_w_spec = pl.BlockSpec((FEATS, FEATS), lambda i: (0, 0))
_b_spec = pl.BlockSpec((FEATS,), lambda i: (0,))
_out_shape = jax.ShapeDtypeStruct((N_PAD, FEATS), jnp.float32)


def _prep_call(f_p, m_p, deg):
    return pl.pallas_call(
        _prep_body,
        grid=(_GRID,),
        in_specs=[_row_spec, _row_spec, _part_spec],
        out_specs=_row_spec,
        out_shape=_out_shape,
    )(f_p, m_p, deg)


def _layer1_call(agg, deg, w, b):
    return pl.pallas_call(
        _layer1_body,
        grid=(_GRID,),
        in_specs=[_part_spec, _part_spec, _w_spec, _b_spec],
        out_specs=_row_spec,
        out_shape=_out_shape,
    )(agg, deg, w, b)


def _layer2_call(agg, deg, w, b):
    return pl.pallas_call(
        _layer2_body,
        grid=(_GRID,),
        in_specs=[_part_spec, _part_spec, _w_spec, _b_spec],
        out_specs=_row_spec,
        out_shape=_out_shape,
    )(agg, deg, w, b)


def kernel(features, mask, edge_index, W1, b1, W2, b2):
    n = features.shape[0]
    src = edge_index[0].astype(jnp.int32)
    dst = edge_index[1].astype(jnp.int32)
    e = src.shape[0]
    e_pad = -(-e // (NW * CHUNK)) * (NW * CHUNK)
    pad = jnp.full((e_pad - e,), n, dtype=jnp.int32)
    src_p = jnp.concatenate([src, pad])
    dst_p = jnp.concatenate([dst, pad])
    f_p = jnp.pad(features, ((0, N_PAD - n), (0, 0)))
    m_p = jnp.pad(mask, ((0, N_PAD - n), (0, 0)))

    deg_kernel = _make_deg_kernel(e_pad)
    agg_kernel = _make_agg_kernel(e_pad)

    ones128 = jnp.ones((CHUNK, FEATS), jnp.float32)
    zeros128 = jnp.zeros((CHUNK, FEATS), jnp.float32)

    deg = deg_kernel(src_p, dst_p, ones128, zeros128)
    h1 = _prep_call(f_p, m_p, deg)
    agg1 = agg_kernel(h1, src_p, dst_p, zeros128)
    h2 = _layer1_call(agg1, deg, W1, b1)
    agg2 = agg_kernel(h2, src_p, dst_p, zeros128)
    out = _layer2_call(agg2, deg, W2, b2)
    return out[:n]
